# baseline (speedup 1.0000x reference)
import functools

import jax
import jax.numpy as jnp
from jax import lax
from jax.experimental import pallas as pl
from jax.experimental.pallas import tpu as pltpu
from jax.experimental.pallas import tpu_sc as plsc

N = 10000
NNZ = 320000
M = 20000
NFEAT = 128
HIDDEN = 128
NCLASS = 40

NC = 2
NS = 16
CH = 160
NNZP = NS * CH * 128
NT = N + 4
MT = 20480
NV = 10240
RB = 64
IB = 32

STARTS_E = [64 * b for b in range(20)]
STARTS_V = [64 * b for b in range(10)]

F32 = jnp.float32
I32 = jnp.int32


def _mesh():
    return plsc.VectorSubcoreMesh(
        core_axis_name="c", subcore_axis_name="s", num_cores=NC, num_subcores=NS
    )


def _zero_stage(stage, rows, feats):
    def zrow(r, _):
        def zcol(c, _):
            stage[r, pl.ds(c * 16, 16)] = jnp.zeros((16,), F32)
            return 0
        return lax.fori_loop(0, feats // 16, zcol, 0)
    lax.fori_loop(0, rows, zrow, 0)


def _fill_ones(buf, rows):
    def frow(r, _):
        buf[r, pl.ds(0, 16)] = jnp.ones((16,), F32)
        return 0
    lax.fori_loop(0, rows, frow, 0)


def _make_cnt():
    RT = MT // NS
    scratch = [
        pltpu.VMEM_SHARED((MT, 16), F32),
        pltpu.VMEM((IB, 128), I32),
        pltpu.VMEM((128, 16), F32),
        pltpu.VMEM((RB, 16), F32),
    ]

    def body(sidx, cnt_out, cnt, eidx, ones16, cbuf):
        s = lax.axis_index("s")
        h = lax.axis_index("c")
        _zero_stage(cbuf, RB, 16)
        _fill_ones(ones16, 128)
        for st in STARTS_E:
            pltpu.sync_copy(cbuf, cnt.at[pl.ds(s * RT + st, RB)])
        plsc.subcore_barrier()

        for bi in range(CH // IB):
            pltpu.sync_copy(sidx.at[pl.ds(s * CH + bi * IB, IB)], eidx)

            def chunk(j, _):
                pltpu.sync_copy(ones16, cnt.at[eidx.at[j]], add=True)
                return 0
            lax.fori_loop(0, IB, chunk, 0)
        plsc.subcore_barrier()

        @pl.when(h == 0)
        def _():
            for st in STARTS_E:
                pltpu.sync_copy(cnt.at[pl.ds(s * RT + st, RB)], cbuf)
                pltpu.sync_copy(cbuf, cnt_out.at[pl.ds(s * RT + st, RB)])

    return pl.kernel(
        body, out_type=jax.ShapeDtypeStruct((MT, 16), F32), mesh=_mesh(),
        scratch_types=scratch,
        compiler_params=pltpu.CompilerParams(use_tc_tiling_on_sc=False),
        name="edge_cnt",
    )


def _make_edge_agg(F):
    RT = MT // NS
    NB = len(STARTS_E)

    scratch = [
        pltpu.VMEM_SHARED((MT, F), F32),
        pltpu.VMEM((IB, 128), I32),
        pltpu.VMEM((IB, 128), I32),
        pltpu.VMEM((4, 128, F), F32),
        pltpu.VMEM((RB, F), F32),
        pltpu.VMEM((128,), F32),
        pltpu.VMEM((RB, 16), F32),
        [pltpu.SemaphoreType.DMA] * 4,
        [pltpu.SemaphoreType.DMA] * 4,
    ]

    def body(table, gidx, sidx, aux, cnt_io, xe, acc, vidx, eidx, rows,
             stage, auxbuf, cbuf, gsem, ssem):
        h = lax.axis_index("c")
        s = lax.axis_index("s")

        _zero_stage(stage, RB, F)
        for st in STARTS_E:
            pltpu.sync_copy(stage, acc.at[pl.ds(s * RT + st, RB)])
        plsc.subcore_barrier()

        for bi in range(CH // IB):
            pltpu.sync_copy(
                gidx.at[pl.ds((h * NS + s) * CH + bi * IB, IB)], vidx)
            pltpu.sync_copy(sidx.at[pl.ds(s * CH + bi * IB, IB)], eidx)

            pltpu.async_copy(table.at[vidx.at[0]], rows.at[0], gsem[0])
            pltpu.async_copy(table.at[vidx.at[1]], rows.at[1], gsem[1])

            def group(k, _):
                for b in range(4):
                    j = 4 * k + b
                    b2 = (b + 2) % 4
                    pltpu.make_async_copy(table.at[pl.ds(0, 128)],
                                          rows.at[b], gsem[b]).wait()
                    pltpu.async_copy(rows.at[b], acc.at[eidx.at[j]],
                                     ssem[b], add=True)

                    @pl.when(j + 2 < IB)
                    def _():
                        @pl.when(j >= 2)
                        def _():
                            pltpu.make_async_copy(
                                rows.at[b2], acc.at[pl.ds(0, 128)],
                                ssem[b2]).wait()
                        pltpu.async_copy(table.at[vidx.at[j + 2]],
                                         rows.at[b2], gsem[b2])
                return 0
            lax.fori_loop(0, IB // 4, group, 0)
            for b in range(4):
                pltpu.make_async_copy(rows.at[b], acc.at[pl.ds(0, 128)],
                                      ssem[b]).wait()
        plsc.subcore_barrier()

        for b, st in enumerate(STARTS_E):
            r0 = s * RT + st
            blockrow = s * NB + b
            pltpu.sync_copy(acc.at[pl.ds(r0, RB)], stage)
            pltpu.sync_copy(cnt_io.at[pl.ds(r0, RB)], cbuf)
            pltpu.sync_copy(aux.at[blockrow], auxbuf)

            def sgroup(q, _):
                av = auxbuf[pl.ds(q * 16, 16)]
                ones = jnp.ones((16,), F32)
                zeros = jnp.zeros((16,), F32)
                for i in range(16):
                    cvec = cbuf[q * 16 + i, pl.ds(0, 16)]
                    avec = jnp.full((16,), av[i], F32)
                    svec = jnp.where(cvec > 0.0,
                                     avec / jnp.maximum(cvec, ones), zeros)

                    def scol(cc, _):
                        stage[q * 16 + i, pl.ds(cc * 16, 16)] = (
                            stage[q * 16 + i, pl.ds(cc * 16, 16)] * svec
                        )
                        return 0
                    lax.fori_loop(0, F // 16, scol, 0)
                return 0
            lax.fori_loop(0, RB // 16, sgroup, 0)

            pltpu.sync_copy(stage, xe.at[h, pl.ds(r0, RB)])

    return pl.kernel(
        body, out_type=jax.ShapeDtypeStruct((2, MT, F), F32),
        mesh=_mesh(), scratch_types=scratch,
        compiler_params=pltpu.CompilerParams(use_tc_tiling_on_sc=False),
        name=f"edge_agg_f{F}",
    )


def _make_vert_agg(F, relu):
    RT = NV // NS
    NB = len(STARTS_V)

    scratch = [
        pltpu.VMEM_SHARED((NV, F), F32),
        pltpu.VMEM((IB, 128), I32),
        pltpu.VMEM((IB, 128), I32),
        pltpu.VMEM((4, 128, F), F32),
        pltpu.VMEM((RB, F), F32),
        pltpu.VMEM((128,), F32),
        [pltpu.SemaphoreType.DMA] * 4,
        [pltpu.SemaphoreType.DMA] * 4,
    ]

    def body(xe, gidx, sidx, degv, out, acc, eidx, vidx, rows, stage,
             auxbuf, gsem, ssem):
        h = lax.axis_index("c")
        s = lax.axis_index("s")

        _zero_stage(stage, RB, F)
        for st in STARTS_V:
            pltpu.sync_copy(stage, acc.at[pl.ds(s * RT + st, RB)])
        plsc.subcore_barrier()

        for bi in range(CH // IB):
            pltpu.sync_copy(
                gidx.at[pl.ds((h * NS + s) * CH + bi * IB, IB)], eidx)
            pltpu.sync_copy(sidx.at[pl.ds(s * CH + bi * IB, IB)], vidx)

            pltpu.async_copy(xe.at[eidx.at[0]], rows.at[0], gsem[0])
            pltpu.async_copy(xe.at[eidx.at[1]], rows.at[1], gsem[1])

            def group(k, _):
                for b in range(4):
                    j = 4 * k + b
                    b2 = (b + 2) % 4
                    pltpu.make_async_copy(xe.at[pl.ds(0, 128)],
                                          rows.at[b], gsem[b]).wait()
                    pltpu.async_copy(rows.at[b], acc.at[vidx.at[j]],
                                     ssem[b], add=True)

                    @pl.when(j + 2 < IB)
                    def _():
                        @pl.when(j >= 2)
                        def _():
                            pltpu.make_async_copy(
                                rows.at[b2], acc.at[pl.ds(0, 128)],
                                ssem[b2]).wait()
                        pltpu.async_copy(xe.at[eidx.at[j + 2]],
                                         rows.at[b2], gsem[b2])
                return 0
            lax.fori_loop(0, IB // 4, group, 0)
            for b in range(4):
                pltpu.make_async_copy(rows.at[b], acc.at[pl.ds(0, 128)],
                                      ssem[b]).wait()
        plsc.subcore_barrier()

        for b, st in enumerate(STARTS_V):
            r0 = s * RT + st
            pltpu.sync_copy(acc.at[pl.ds(r0, RB)], stage)
            pltpu.sync_copy(degv.at[s * NB + b], auxbuf)

            def sgroup(q, _):
                scv = auxbuf[pl.ds(q * 16, 16)]
                for i in range(16):
                    sc = scv[i]

                    def scol(cc, _):
                        v = (stage[q * 16 + i, pl.ds(cc * 16, 16)]
                             * jnp.full((16,), sc, F32))
                        if relu:
                            v = jnp.maximum(v, jnp.zeros((16,), F32))
                        stage[q * 16 + i, pl.ds(cc * 16, 16)] = v
                        return 0
                    lax.fori_loop(0, F // 16, scol, 0)
                return 0
            lax.fori_loop(0, RB // 16, sgroup, 0)

            pltpu.sync_copy(stage, out.at[h, pl.ds(r0, RB)])

    return pl.kernel(
        body, out_type=jax.ShapeDtypeStruct((2, NV, F), F32), mesh=_mesh(),
        scratch_types=scratch,
        compiler_params=pltpu.CompilerParams(use_tc_tiling_on_sc=False),
        name=f"vert_agg_f{F}" + ("_relu" if relu else ""),
    )


def _mm1_body(x_ref, w_ref, out_ref):
    p = jnp.dot(x_ref[...], w_ref[...], preferred_element_type=F32)
    out_ref[0, pl.ds(0, N), :] = p[:, :64]
    out_ref[1, pl.ds(0, N), :] = p[:, 64:]


def _mm2_body(h_ref, w_ref, out_ref):
    hf = jnp.concatenate([h_ref[0, pl.ds(0, N), :], h_ref[1, pl.ds(0, N), :]],
                         axis=-1)
    p = jnp.dot(hf, w_ref[...], preferred_element_type=F32)
    out_ref[0, pl.ds(0, N), :] = p[:, :32]
    out_ref[1, pl.ds(0, N), :] = jnp.concatenate(
        [p[:, 32:40], jnp.zeros((N, 24), F32)], axis=-1)


def _logsoftmax_body(o_ref, out_ref):
    logits = jnp.concatenate(
        [o_ref[0, pl.ds(0, N), :], o_ref[1, pl.ds(0, N), pl.ds(0, 8)]],
        axis=-1)
    m = jnp.max(logits, axis=1, keepdims=True)
    e = jnp.exp(logits - m)
    lse = jnp.log(jnp.sum(e, axis=1, keepdims=True))
    out_ref[...] = logits - m - lse


_edge_cnt = _make_cnt()
_edge_agg64 = _make_edge_agg(64)
_edge_agg32 = _make_edge_agg(32)
_vert_agg64 = _make_vert_agg(64, relu=True)
_vert_agg32 = _make_vert_agg(32, relu=False)

_mm1 = pl.pallas_call(
    _mm1_body, out_shape=jax.ShapeDtypeStruct((2, NT, 64), F32))
_mm2 = pl.pallas_call(
    _mm2_body, out_shape=jax.ShapeDtypeStruct((2, NT, 32), F32))
_logsoftmax = pl.pallas_call(
    _logsoftmax_body, out_shape=jax.ShapeDtypeStruct((N, NCLASS), F32))


@jax.jit
def kernel(X, vertex, edges, degE, degV, W1, W2):
    vertex = vertex.astype(I32)
    edges = edges.astype(I32)
    pad = NNZP - NNZ
    vp = jnp.concatenate([vertex, jnp.full((pad,), N, I32)])
    ep = jnp.concatenate([edges, jnp.full((pad,), M, I32)])
    vg = jnp.concatenate([vp, vp + NT]).reshape(2 * NS * CH, 128)
    eg = jnp.concatenate([ep, ep + MT]).reshape(2 * NS * CH, 128)
    vs = vp.reshape(NS * CH, 128)
    es = ep.reshape(NS * CH, 128)
    lanes = jnp.arange(128, dtype=I32)
    rows_e = (jnp.arange(NS, dtype=I32)[:, None, None] * (MT // NS)
              + jnp.array(STARTS_E, I32)[None, :, None] + lanes[None, None, :])
    rows_v = (jnp.arange(NS, dtype=I32)[:, None, None] * (NV // NS)
              + jnp.array(STARTS_V, I32)[None, :, None] + lanes[None, None, :])
    degE2 = degE.reshape(-1)[rows_e].reshape(NS * len(STARTS_E), 128)
    degV2 = degV.reshape(-1)[rows_v].reshape(NS * len(STARTS_V), 128)

    cnt = _edge_cnt(es)
    t1 = _mm1(X, W1)
    xe = _edge_agg64(t1.reshape(2 * NT, 64), vg, es, degE2, cnt)
    hmat = _vert_agg64(xe.reshape(2 * MT, 64), eg, vs, degV2)
    t2 = _mm2(hmat, W2)
    xe2 = _edge_agg32(t2.reshape(2 * NT, 32), vg, es, degE2, cnt)
    o = _vert_agg32(xe2.reshape(2 * MT, 32), eg, vs, degV2)
    return _logsoftmax(o)


# speedup vs baseline: 1.4548x; 1.4548x over previous
import functools

import jax
import jax.numpy as jnp
from jax import lax
from jax.experimental import pallas as pl
from jax.experimental.pallas import tpu as pltpu
from jax.experimental.pallas import tpu_sc as plsc

N = 10000
NNZ = 320000
M = 20000
NFEAT = 128
HIDDEN = 128
NCLASS = 40

NC = 2
NS = 16
CH = 160
NNZP = NS * CH * 128
NT = N + 4
MT = 20480
NV = 10240
RB = 128
IB = 16

STARTS_E = [128 * b for b in range(10)]
STARTS_V = [128 * b for b in range(5)]

F32 = jnp.float32
I32 = jnp.int32


def _mesh():
    return plsc.VectorSubcoreMesh(
        core_axis_name="c", subcore_axis_name="s", num_cores=NC, num_subcores=NS
    )


def _zero_stage(stage, rows, feats):
    def zrow(r, _):
        def zcol(c, _):
            stage[r, pl.ds(c * 16, 16)] = jnp.zeros((16,), F32)
            return 0
        return lax.fori_loop(0, feats // 16, zcol, 0)
    lax.fori_loop(0, rows, zrow, 0)


def _fill_ones(buf, rows):
    def frow(r, _):
        buf[r, pl.ds(0, 16)] = jnp.ones((16,), F32)
        return 0
    lax.fori_loop(0, rows, frow, 0)


def _make_cnt():
    RT = MT // NS
    scratch = [
        pltpu.VMEM_SHARED((MT, 16), F32),
        pltpu.VMEM((IB, 128), I32),
        pltpu.VMEM((128, 16), F32),
        pltpu.VMEM((RB, 16), F32),
    ]

    def body(sidx, cnt_out, cnt, eidx, ones16, cbuf):
        s = lax.axis_index("s")
        h = lax.axis_index("c")
        _zero_stage(cbuf, RB, 16)
        _fill_ones(ones16, 128)
        for st in STARTS_E:
            pltpu.sync_copy(cbuf, cnt.at[pl.ds(s * RT + st, RB)])
        plsc.subcore_barrier()

        for bi in range(CH // IB):
            pltpu.sync_copy(sidx.at[pl.ds(s * CH + bi * IB, IB)], eidx)

            def chunk(j, _):
                pltpu.sync_copy(ones16, cnt.at[eidx.at[j]], add=True)
                return 0
            lax.fori_loop(0, IB, chunk, 0)
        plsc.subcore_barrier()

        @pl.when(h == 0)
        def _():
            for st in STARTS_E:
                pltpu.sync_copy(cnt.at[pl.ds(s * RT + st, RB)], cbuf)
                pltpu.sync_copy(cbuf, cnt_out.at[pl.ds(s * RT + st, RB)])

    return pl.kernel(
        body, out_type=jax.ShapeDtypeStruct((MT, 16), F32), mesh=_mesh(),
        scratch_types=scratch,
        compiler_params=pltpu.CompilerParams(use_tc_tiling_on_sc=False),
        name="edge_cnt",
    )


def _make_edge_agg(F):
    RT = MT // NS
    NB = len(STARTS_E)

    scratch = [
        pltpu.VMEM_SHARED((MT, F), F32),
        pltpu.VMEM((IB, 128), I32),
        pltpu.VMEM((IB, 128), I32),
        pltpu.VMEM((4, 128, F), F32),
        pltpu.VMEM((RB, F), F32),
        pltpu.VMEM((128,), F32),
        pltpu.VMEM((RB, 16), F32),
        [pltpu.SemaphoreType.DMA] * 4,
    ]

    def body(table, gidx, sidx, aux, cnt_io, xe, acc, vidx, eidx, rows,
             stage, auxbuf, cbuf, gsem):
        h = lax.axis_index("c")
        s = lax.axis_index("s")

        _zero_stage(stage, RB, F)
        for st in STARTS_E:
            pltpu.sync_copy(stage, acc.at[pl.ds(s * RT + st, RB)])
        plsc.subcore_barrier()

        for bi in range(CH // IB):
            pltpu.sync_copy(
                gidx.at[pl.ds((h * NS + s) * CH + bi * IB, IB)], vidx)
            pltpu.sync_copy(sidx.at[pl.ds(s * CH + bi * IB, IB)], eidx)

            for b in range(4):
                pltpu.async_copy(table.at[vidx.at[b]], rows.at[b], gsem[b])

            def group(k, _):
                for b in range(4):
                    j = 4 * k + b

                    @pl.when(j + 4 < IB)
                    def _():
                        pltpu.async_copy(table.at[vidx.at[j + 4]],
                                         rows.at[b], gsem[b])
                    pltpu.make_async_copy(table.at[pl.ds(0, 128)],
                                          rows.at[b], gsem[b]).wait()
                    pltpu.sync_copy(rows.at[b], acc.at[eidx.at[j]],
                                    add=True)
                return 0
            lax.fori_loop(0, IB // 4, group, 0)
        plsc.subcore_barrier()

        for b, st in enumerate(STARTS_E):
            r0 = s * RT + st
            blockrow = s * NB + b
            pltpu.sync_copy(acc.at[pl.ds(r0, RB)], stage)
            pltpu.sync_copy(cnt_io.at[pl.ds(r0, RB)], cbuf)
            pltpu.sync_copy(aux.at[blockrow], auxbuf)

            def sgroup(q, _):
                av = auxbuf[pl.ds(q * 16, 16)]
                ones = jnp.ones((16,), F32)
                zeros = jnp.zeros((16,), F32)
                for i in range(16):
                    cvec = cbuf[q * 16 + i, pl.ds(0, 16)]
                    avec = jnp.full((16,), av[i], F32)
                    svec = jnp.where(cvec > 0.0,
                                     avec / jnp.maximum(cvec, ones), zeros)

                    def scol(cc, _):
                        stage[q * 16 + i, pl.ds(cc * 16, 16)] = (
                            stage[q * 16 + i, pl.ds(cc * 16, 16)] * svec
                        )
                        return 0
                    lax.fori_loop(0, F // 16, scol, 0)
                return 0
            lax.fori_loop(0, RB // 16, sgroup, 0)

            pltpu.sync_copy(stage, xe.at[h, pl.ds(r0, RB)])

    return pl.kernel(
        body, out_type=jax.ShapeDtypeStruct((2, MT, F), F32),
        mesh=_mesh(), scratch_types=scratch,
        compiler_params=pltpu.CompilerParams(use_tc_tiling_on_sc=False),
        name=f"edge_agg_f{F}",
    )


def _make_vert_agg(F, relu):
    RT = NV // NS
    NB = len(STARTS_V)

    scratch = [
        pltpu.VMEM_SHARED((NV, F), F32),
        pltpu.VMEM((IB, 128), I32),
        pltpu.VMEM((IB, 128), I32),
        pltpu.VMEM((4, 128, F), F32),
        pltpu.VMEM((RB, F), F32),
        pltpu.VMEM((128,), F32),
        [pltpu.SemaphoreType.DMA] * 4,
    ]

    def body(xe, gidx, sidx, degv, out, acc, eidx, vidx, rows, stage,
             auxbuf, gsem):
        h = lax.axis_index("c")
        s = lax.axis_index("s")

        _zero_stage(stage, RB, F)
        for st in STARTS_V:
            pltpu.sync_copy(stage, acc.at[pl.ds(s * RT + st, RB)])
        plsc.subcore_barrier()

        for bi in range(CH // IB):
            pltpu.sync_copy(
                gidx.at[pl.ds((h * NS + s) * CH + bi * IB, IB)], eidx)
            pltpu.sync_copy(sidx.at[pl.ds(s * CH + bi * IB, IB)], vidx)

            for b in range(4):
                pltpu.async_copy(xe.at[eidx.at[b]], rows.at[b], gsem[b])

            def group(k, _):
                for b in range(4):
                    j = 4 * k + b

                    @pl.when(j + 4 < IB)
                    def _():
                        pltpu.async_copy(xe.at[eidx.at[j + 4]],
                                         rows.at[b], gsem[b])
                    pltpu.make_async_copy(xe.at[pl.ds(0, 128)],
                                          rows.at[b], gsem[b]).wait()
                    pltpu.sync_copy(rows.at[b], acc.at[vidx.at[j]],
                                    add=True)
                return 0
            lax.fori_loop(0, IB // 4, group, 0)
        plsc.subcore_barrier()

        for b, st in enumerate(STARTS_V):
            r0 = s * RT + st
            pltpu.sync_copy(acc.at[pl.ds(r0, RB)], stage)
            pltpu.sync_copy(degv.at[s * NB + b], auxbuf)

            def sgroup(q, _):
                scv = auxbuf[pl.ds(q * 16, 16)]
                for i in range(16):
                    sc = scv[i]

                    def scol(cc, _):
                        v = (stage[q * 16 + i, pl.ds(cc * 16, 16)]
                             * jnp.full((16,), sc, F32))
                        if relu:
                            v = jnp.maximum(v, jnp.zeros((16,), F32))
                        stage[q * 16 + i, pl.ds(cc * 16, 16)] = v
                        return 0
                    lax.fori_loop(0, F // 16, scol, 0)
                return 0
            lax.fori_loop(0, RB // 16, sgroup, 0)

            pltpu.sync_copy(stage, out.at[h, pl.ds(r0, RB)])

    return pl.kernel(
        body, out_type=jax.ShapeDtypeStruct((2, NV, F), F32), mesh=_mesh(),
        scratch_types=scratch,
        compiler_params=pltpu.CompilerParams(use_tc_tiling_on_sc=False),
        name=f"vert_agg_f{F}" + ("_relu" if relu else ""),
    )


def _mm1_body(x_ref, w_ref, out_ref):
    p = jnp.dot(x_ref[...], w_ref[...], preferred_element_type=F32)
    out_ref[0, pl.ds(0, N), :] = p[:, :64]
    out_ref[1, pl.ds(0, N), :] = p[:, 64:]


def _mm2_body(h_ref, w_ref, out_ref):
    hf = jnp.concatenate([h_ref[0, pl.ds(0, N), :], h_ref[1, pl.ds(0, N), :]],
                         axis=-1)
    p = jnp.dot(hf, w_ref[...], preferred_element_type=F32)
    out_ref[0, pl.ds(0, N), :] = p[:, :32]
    out_ref[1, pl.ds(0, N), :] = jnp.concatenate(
        [p[:, 32:40], jnp.zeros((N, 24), F32)], axis=-1)


def _logsoftmax_body(o_ref, out_ref):
    logits = jnp.concatenate(
        [o_ref[0, pl.ds(0, N), :], o_ref[1, pl.ds(0, N), pl.ds(0, 8)]],
        axis=-1)
    m = jnp.max(logits, axis=1, keepdims=True)
    e = jnp.exp(logits - m)
    lse = jnp.log(jnp.sum(e, axis=1, keepdims=True))
    out_ref[...] = logits - m - lse


_edge_cnt = _make_cnt()
_edge_agg64 = _make_edge_agg(64)
_edge_agg32 = _make_edge_agg(32)
_vert_agg64 = _make_vert_agg(64, relu=True)
_vert_agg32 = _make_vert_agg(32, relu=False)

_mm1 = pl.pallas_call(
    _mm1_body, out_shape=jax.ShapeDtypeStruct((2, NT, 64), F32))
_mm2 = pl.pallas_call(
    _mm2_body, out_shape=jax.ShapeDtypeStruct((2, NT, 32), F32))
_logsoftmax = pl.pallas_call(
    _logsoftmax_body, out_shape=jax.ShapeDtypeStruct((N, NCLASS), F32))


@jax.jit
def kernel(X, vertex, edges, degE, degV, W1, W2):
    vertex = vertex.astype(I32)
    edges = edges.astype(I32)
    pad = NNZP - NNZ
    vp = jnp.concatenate([vertex, jnp.full((pad,), N, I32)])
    ep = jnp.concatenate([edges, jnp.full((pad,), M, I32)])
    vg = jnp.concatenate([vp, vp + NT]).reshape(2 * NS * CH, 128)
    eg = jnp.concatenate([ep, ep + MT]).reshape(2 * NS * CH, 128)
    vs = vp.reshape(NS * CH, 128)
    es = ep.reshape(NS * CH, 128)
    lanes = jnp.arange(128, dtype=I32)
    rows_e = (jnp.arange(NS, dtype=I32)[:, None, None] * (MT // NS)
              + jnp.array(STARTS_E, I32)[None, :, None] + lanes[None, None, :])
    rows_v = (jnp.arange(NS, dtype=I32)[:, None, None] * (NV // NS)
              + jnp.array(STARTS_V, I32)[None, :, None] + lanes[None, None, :])
    degE2 = degE.reshape(-1)[rows_e].reshape(NS * len(STARTS_E), 128)
    degV2 = degV.reshape(-1)[rows_v].reshape(NS * len(STARTS_V), 128)

    cnt = _edge_cnt(es)
    t1 = _mm1(X, W1)
    xe = _edge_agg64(t1.reshape(2 * NT, 64), vg, es, degE2, cnt)
    hmat = _vert_agg64(xe.reshape(2 * MT, 64), eg, vs, degV2)
    t2 = _mm2(hmat, W2)
    xe2 = _edge_agg32(t2.reshape(2 * NT, 32), vg, es, degE2, cnt)
    o = _vert_agg32(xe2.reshape(2 * MT, 32), eg, vs, degV2)
    return _logsoftmax(o)
